# baseline (speedup 1.0000x reference)
import functools

import jax
import jax.numpy as jnp
from jax import lax
from jax.experimental import pallas as pl
from jax.experimental.pallas import tpu as pltpu
from jax.experimental.pallas import tpu_sc as plsc

VOCAB = 1000000
DIM = 64
B = 4096
L = 50

N = B * L
NC, NS = 2, 16
NW = NC * NS
PER_W = N // NW
CHUNK = 160
NCHUNK = PER_W // CHUNK
NGROUP = CHUNK // 16


def _gather_kernel(emb2_hbm, idxp_hbm, par_hbm, out_hbm,
                   idxp_v, par_v, pairs_v, out_v, sem):
    wid = lax.axis_index("s") * NC + lax.axis_index("c")
    base = wid * PER_W
    iota16 = jax.lax.iota(jnp.int32, 16)

    def chunk_body(j, _):
        off = pl.multiple_of(base + j * CHUNK, CHUNK)
        pltpu.sync_copy(idxp_hbm.at[pl.ds(off, CHUNK)], idxp_v)
        pltpu.sync_copy(par_hbm.at[pl.ds(off, CHUNK)], par_v)
        pltpu.async_copy(emb2_hbm.at[idxp_v], pairs_v, sem).wait()

        def group_body(g, _):
            row16 = iota16 + g * 16
            par16 = par_v[pl.ds(g * 16, 16)]
            for d in range(DIM):
                col = par16 + d
                val = plsc.load_gather(pairs_v, [row16, col])
                dcol = jnp.full((16,), d, jnp.int32)
                plsc.store_scatter(out_v, [row16, dcol], val)
            return 0

        lax.fori_loop(0, NGROUP, group_body, 0)
        pltpu.sync_copy(out_v, out_hbm.at[pl.ds(off, CHUNK)])
        return 0

    lax.fori_loop(0, NCHUNK, chunk_body, 0)


@jax.jit
def _embed_lookup(emb2, idxp, par64):
    mesh = plsc.VectorSubcoreMesh(core_axis_name="c", subcore_axis_name="s")
    k = pl.kernel(
        _gather_kernel,
        mesh=mesh,
        compiler_params=pltpu.CompilerParams(
            use_tc_tiling_on_sc=True, needs_layout_passes=False),
        out_type=jax.ShapeDtypeStruct((N, DIM), jnp.float32),
        scratch_types=[
            pltpu.VMEM((CHUNK,), jnp.int32),
            pltpu.VMEM((CHUNK,), jnp.int32),
            pltpu.VMEM((CHUNK, 2 * DIM), jnp.float32),
            pltpu.VMEM((CHUNK, DIM), jnp.float32),
            pltpu.SemaphoreType.DMA,
        ],
    )
    return k(emb2, idxp, par64)


def kernel(emb, inputs):
    emb2 = emb.reshape(VOCAB // 2, 2 * DIM)
    idx = inputs.reshape(N)
    idxp = idx >> 1
    par64 = (idx & 1) << 6
    out = _embed_lookup(emb2, idxp, par64)
    return out.reshape(B, L, DIM)
